# Initial kernel scaffold; baseline (speedup 1.0000x reference)
import jax
import jax.numpy as jnp
from jax.experimental import pallas as pl


def kernel(x, edge_attr, W1a, b1a, W1b, b1b, eps1, W2a, b2a, W2b, b2b, eps2, W3a, b3a, W3b, b3b, eps3, Wfc, edge_index, batch):
    raise NotImplementedError("write your pallas kernel here")


# speedup vs baseline: 4.0620x; 4.0620x over previous
import functools

import jax
import jax.numpy as jnp
from jax import lax
from jax.experimental import pallas as pl
from jax.experimental.pallas import tpu as pltpu, tpu_sc as plsc

_N = 10000
_E = 320000
_H = 256
_HH = 128
_G = 64
_NC = 2
_NS = 16
_CHUNK = 128
_ROWS_SUB = 624
_ROWS_TAIL = _N - _NS * _ROWS_SUB
_ACC_PAD = _N + 8

@functools.lru_cache(maxsize=None)
def _sc_mesh():
    return plsc.VectorSubcoreMesh(core_axis_name="c", subcore_axis_name="s",
                                  num_cores=_NC, num_subcores=_NS)


def _fill_i32(ref_row, start, n, value):
    for j in range(start // 16, (start + n) // 16):
        ref_row[pl.ds(j * 16, 16)] = jnp.full((16,), value, jnp.int32)


@functools.lru_cache(maxsize=None)
def _sc_agg_big_call():
    return pl.kernel(
        _sc_agg_big,
        out_type=jax.ShapeDtypeStruct((2 * _N, _HH), jnp.float32),
        mesh=_sc_mesh(),
        scratch_types=[
            pltpu.VMEM((1, _CHUNK), jnp.int32),
            pltpu.VMEM((1, _CHUNK), jnp.int32),
            pltpu.VMEM((_CHUNK, _HH), jnp.float32),
            pltpu.VMEM_SHARED((_ACC_PAD, _HH), jnp.float32),
            pltpu.SemaphoreType.DMA,
        ],
    )


def _sc_agg_big(h_hbm, src_hbm, dst_hbm, out_hbm, sidx, didx, rows, acc, gsem):
    c = lax.axis_index("c")
    s = lax.axis_index("s")
    row0 = s * _ROWS_SUB
    coff = c * _N
    pltpu.sync_copy(h_hbm.at[pl.ds(coff + row0, _ROWS_SUB)],
                    acc.at[pl.ds(row0, _ROWS_SUB)])

    @pl.when(s == _NS - 1)
    def _():
        pltpu.sync_copy(h_hbm.at[pl.ds(coff + _NS * _ROWS_SUB, _ROWS_TAIL)],
                        acc.at[pl.ds(_NS * _ROWS_SUB, _ROWS_TAIL)])

    plsc.subcore_barrier()

    epw = _E // _NS
    base = s * epw
    nfull = epw // _CHUNK
    rem = epw - nfull * _CHUNK

    def chunk(off, k_real):
        pltpu.sync_copy(src_hbm.at[pl.ds(off, k_real)],
                        sidx.at[0, pl.ds(0, k_real)])
        pltpu.sync_copy(dst_hbm.at[pl.ds(off, k_real)],
                        didx.at[0, pl.ds(0, k_real)])
        if k_real < _CHUNK:
            _fill_i32(sidx.at[0], k_real, _CHUNK - k_real, 0)
            _fill_i32(didx.at[0], k_real, _CHUNK - k_real, _N)
        for j in range(_CHUNK // 16):
            sl = pl.ds(j * 16, 16)
            sidx[0, sl] = sidx[0, sl] + coff
        pltpu.async_copy(h_hbm.at[sidx.at[0]], rows, gsem).wait()
        pltpu.sync_copy(rows, acc.at[didx.at[0]], add=True)

    def body(i, _):
        chunk(base + i * _CHUNK, _CHUNK)
        return 0

    lax.fori_loop(0, nfull, body, 0)
    chunk(base + nfull * _CHUNK, rem)

    plsc.subcore_barrier()
    pltpu.sync_copy(acc.at[pl.ds(row0, _ROWS_SUB)],
                    out_hbm.at[pl.ds(coff + row0, _ROWS_SUB)])

    @pl.when(s == _NS - 1)
    def _():
        pltpu.sync_copy(acc.at[pl.ds(_NS * _ROWS_SUB, _ROWS_TAIL)],
                        out_hbm.at[pl.ds(coff + _NS * _ROWS_SUB, _ROWS_TAIL)])


@functools.lru_cache(maxsize=None)
def _sc_agg_small_call():
    return pl.kernel(
        _sc_agg_small,
        out_type=jax.ShapeDtypeStruct((_NC, _N, _HH), jnp.float32),
        mesh=_sc_mesh(),
        scratch_types=[
            pltpu.VMEM((1, _CHUNK), jnp.int32),
            pltpu.VMEM((1, _CHUNK), jnp.int32),
            pltpu.VMEM((_CHUNK, _HH), jnp.float32),
            pltpu.VMEM_SHARED((_ACC_PAD, _HH), jnp.float32),
            pltpu.SemaphoreType.DMA,
        ],
    )


def _sc_agg_small(x_hbm, src_hbm, dst_hbm, out_hbm, sidx, didx, rows, acc, gsem):
    c = lax.axis_index("c")
    s = lax.axis_index("s")
    row0 = s * _ROWS_SUB
    pltpu.sync_copy(x_hbm.at[pl.ds(row0, _ROWS_SUB)],
                    acc.at[pl.ds(row0, _ROWS_SUB)])

    @pl.when(s == _NS - 1)
    def _():
        pltpu.sync_copy(x_hbm.at[pl.ds(_NS * _ROWS_SUB, _ROWS_TAIL)],
                        acc.at[pl.ds(_NS * _ROWS_SUB, _ROWS_TAIL)])

    plsc.subcore_barrier()

    epw = _E // (_NC * _NS)
    base = c * (_E // _NC) + s * epw
    nfull = epw // _CHUNK
    rem = epw - nfull * _CHUNK

    def chunk(off, k_real):
        pltpu.sync_copy(src_hbm.at[pl.ds(off, k_real)],
                        sidx.at[0, pl.ds(0, k_real)])
        pltpu.sync_copy(dst_hbm.at[pl.ds(off, k_real)],
                        didx.at[0, pl.ds(0, k_real)])
        if k_real < _CHUNK:
            _fill_i32(sidx.at[0], k_real, _CHUNK - k_real, 0)
            _fill_i32(didx.at[0], k_real, _CHUNK - k_real, _N)
        pltpu.async_copy(x_hbm.at[sidx.at[0]], rows, gsem).wait()
        pltpu.sync_copy(rows, acc.at[didx.at[0]], add=True)

    def body(i, _):
        chunk(base + i * _CHUNK, _CHUNK)
        return 0

    lax.fori_loop(0, nfull, body, 0)
    chunk(base + nfull * _CHUNK, rem)

    plsc.subcore_barrier()
    pltpu.sync_copy(acc.at[pl.ds(row0, _ROWS_SUB)],
                    out_hbm.at[c, pl.ds(row0, _ROWS_SUB)])

    @pl.when(s == _NS - 1)
    def _():
        pltpu.sync_copy(acc.at[pl.ds(_NS * _ROWS_SUB, _ROWS_TAIL)],
                        out_hbm.at[c, pl.ds(_NS * _ROWS_SUB, _ROWS_TAIL)])


_R = 1000
_GRID = _N // _R


def _mlp_tail(t, wa_ref, ba_ref, wb_ref, bb_ref):
    u = lax.dot_general(t, wa_ref[...], (((1,), (1,)), ((), ())),
                        preferred_element_type=jnp.float32,
                        precision=lax.Precision.HIGHEST) + ba_ref[...]
    u = jnp.maximum(u, 0.0)
    return lax.dot_general(u, wb_ref[...], (((1,), (1,)), ((), ())),
                           preferred_element_type=jnp.float32,
                           precision=lax.Precision.HIGHEST) + bb_ref[...]


def _mlp1_body(eps_ref, p_ref, x_ref, wa_ref, ba_ref, wb_ref, bb_ref, o_ref):
    x = x_ref[...]
    agg = p_ref[0][:, :4] + p_ref[1][:, :4] - x
    t = (1.0 + eps_ref[0, 0]) * x + jnp.maximum(agg, 0.0)
    h = _mlp_tail(t, wa_ref, ba_ref, wb_ref, bb_ref)
    o_ref[0] = h[:, :_HH]
    o_ref[1] = h[:, _HH:]


def _mlp_mid_body(eps_ref, s_ref, h_ref, wa_ref, ba_ref, wb_ref, bb_ref, o_ref):
    scale = 1.0 + eps_ref[0, 0]
    t0 = scale * h_ref[0] + jnp.maximum(s_ref[0], 0.0)
    t1 = scale * h_ref[1] + jnp.maximum(s_ref[1], 0.0)
    t = jnp.concatenate([t0, t1], axis=1)
    h = _mlp_tail(t, wa_ref, ba_ref, wb_ref, bb_ref)
    o_ref[0] = h[:, :_HH]
    o_ref[1] = h[:, _HH:]


def _mlp_pool_body(eps_ref, s_ref, h_ref, wa_ref, ba_ref, wb_ref, bb_ref,
                   b_ref, sums_ref):
    @pl.when(pl.program_id(0) == 0)
    def _():
        sums_ref[...] = jnp.zeros_like(sums_ref)

    scale = 1.0 + eps_ref[0, 0]
    t0 = scale * h_ref[0] + jnp.maximum(s_ref[0], 0.0)
    t1 = scale * h_ref[1] + jnp.maximum(s_ref[1], 0.0)
    t = jnp.concatenate([t0, t1], axis=1)
    h = _mlp_tail(t, wa_ref, ba_ref, wb_ref, bb_ref)
    bids = b_ref[0, 0, :]
    onehot = (bids[None, :] == lax.broadcasted_iota(jnp.int32, (_G, _R), 0)
              ).astype(jnp.float32)
    sums_ref[...] += lax.dot_general(onehot, h, (((1,), (0,)), ((), ())),
                                     preferred_element_type=jnp.float32,
                                     precision=lax.Precision.HIGHEST)


def _final_body(sums_ref, b_ref, wfc_ref, o_ref):
    b = b_ref[...]
    onehot = (b[None, :, :] ==
              lax.broadcasted_iota(jnp.int32, (_G,) + b.shape, 0)
              ).astype(jnp.float32)
    counts = jnp.maximum(jnp.sum(onehot, axis=(1, 2)), 1.0)
    pooled = sums_ref[...] / counts[:, None]
    logits = lax.dot_general(pooled, wfc_ref[...], (((1,), (1,)), ((), ())),
                             preferred_element_type=jnp.float32,
                             precision=lax.Precision.HIGHEST)
    o_ref[...] = jax.nn.sigmoid(logits)


_smem11 = pl.BlockSpec((1, 1), lambda i: (0, 0), memory_space=pltpu.SMEM)
_whole = lambda shape: pl.BlockSpec(shape, lambda i: tuple(0 for _ in shape))
_tc_params = pltpu.CompilerParams(dimension_semantics=("arbitrary",))

_mlp1_call = pl.pallas_call(
    _mlp1_body,
    grid=(_GRID,),
    in_specs=[
        _smem11,
        pl.BlockSpec((2, _R, _HH), lambda i: (0, i, 0)),
        pl.BlockSpec((_R, 4), lambda i: (i, 0)),
        _whole((_H, 4)), _whole((1, _H)), _whole((_H, _H)), _whole((1, _H)),
    ],
    out_specs=pl.BlockSpec((2, _R, _HH), lambda i: (0, i, 0)),
    out_shape=jax.ShapeDtypeStruct((2, _N, _HH), jnp.float32),
    compiler_params=_tc_params,
)

_mlp_mid_call = pl.pallas_call(
    _mlp_mid_body,
    grid=(_GRID,),
    in_specs=[
        _smem11,
        pl.BlockSpec((2, _R, _HH), lambda i: (0, i, 0)),
        pl.BlockSpec((2, _R, _HH), lambda i: (0, i, 0)),
        _whole((_H, _H)), _whole((1, _H)), _whole((_H, _H)), _whole((1, _H)),
    ],
    out_specs=pl.BlockSpec((2, _R, _HH), lambda i: (0, i, 0)),
    out_shape=jax.ShapeDtypeStruct((2, _N, _HH), jnp.float32),
    compiler_params=_tc_params,
)

_mlp_pool_call = pl.pallas_call(
    _mlp_pool_body,
    grid=(_GRID,),
    in_specs=[
        _smem11,
        pl.BlockSpec((2, _R, _HH), lambda i: (0, i, 0)),
        pl.BlockSpec((2, _R, _HH), lambda i: (0, i, 0)),
        _whole((_H, _H)), _whole((1, _H)), _whole((_H, _H)), _whole((1, _H)),
        pl.BlockSpec((1, 1, _R), lambda i: (i, 0, 0)),
    ],
    out_specs=pl.BlockSpec((_G, _H), lambda i: (0, 0)),
    out_shape=jax.ShapeDtypeStruct((_G, _H), jnp.float32),
    compiler_params=_tc_params,
)

_final_call = pl.pallas_call(
    _final_body,
    out_shape=jax.ShapeDtypeStruct((_G, 1), jnp.float32),
)


def kernel(x, edge_attr, W1a, b1a, W1b, b1b, eps1, W2a, b2a, W2b, b2b, eps2,
           W3a, b3a, W3b, b3b, eps3, Wfc, edge_index, batch):
    e1 = eps1.reshape(1, 1)
    e2 = eps2.reshape(1, 1)
    e3 = eps3.reshape(1, 1)
    b1a_, b1b_ = b1a.reshape(1, _H), b1b.reshape(1, _H)
    b2a_, b2b_ = b2a.reshape(1, _H), b2b.reshape(1, _H)
    b3a_, b3b_ = b3a.reshape(1, _H), b3b.reshape(1, _H)

    src = edge_index[0]
    dst = edge_index[1]
    x_pad = jnp.pad(x, ((0, 0), (0, _HH - 4)))

    p = _sc_agg_small_call()(x_pad, src, dst)
    h1 = _mlp1_call(e1, p, x, W1a, b1a_, W1b, b1b_)

    s2 = _sc_agg_big_call()(h1.reshape(2 * _N, _HH), src, dst)
    h2 = _mlp_mid_call(e2, s2.reshape(2, _N, _HH), h1, W2a, b2a_, W2b, b2b_)

    s3 = _sc_agg_big_call()(h2.reshape(2 * _N, _HH), src, dst)
    sums = _mlp_pool_call(e3, s3.reshape(2, _N, _HH), h2, W3a, b3a_, W3b,
                          b3b_, batch.reshape(_GRID, 1, _R))

    return _final_call(sums, batch.reshape(8, _N // 8), Wfc)


# speedup vs baseline: 6.2510x; 1.5389x over previous
import functools

import jax
import jax.numpy as jnp
from jax import lax
from jax.experimental import pallas as pl
from jax.experimental.pallas import tpu as pltpu, tpu_sc as plsc

_N = 10000
_E = 320000
_H = 256
_HH = 128
_G = 64
_NC = 2
_NS = 16
_CHUNK = 128
_ROWS_SUB = 624
_ROWS_TAIL = _N - _NS * _ROWS_SUB
_ACC_PAD = _N + 8

@functools.lru_cache(maxsize=None)
def _sc_mesh():
    return plsc.VectorSubcoreMesh(core_axis_name="c", subcore_axis_name="s",
                                  num_cores=_NC, num_subcores=_NS)


_CH = 128
_GE = _CH
_IDX_PAD = 512


def _sc_agg_body(big, h_hbm, src_hbm, dst_hbm, out_hbm,
                 sbuf0, sbuf1, dbuf, rbuf, acc,
                 isem0, isem1, gsem0, gsem1):
    c = lax.axis_index("c")
    s = lax.axis_index("s")
    isems = (isem0, isem1)
    gsems = (gsem0, gsem1)
    sbufs = (sbuf0, sbuf1)
    if big:
        epw = _E // _NS
        src_base = c * _E + s * epw
        dst_base = s * epw
        roff = c * _N
        fillv = c * _N
    else:
        epw = _E // (_NC * _NS)
        src_base = c * (_E // _NC) + s * epw
        dst_base = src_base
        roff = 0
        fillv = 0
    ngrp = epw // _GE
    rem = epw - ngrp * _GE

    def out_at(lo, n):
        if big:
            return out_hbm.at[pl.ds(roff + lo, n)]
        return out_hbm.at[c, pl.ds(lo, n)]

    row0 = s * _ROWS_SUB
    pltpu.sync_copy(h_hbm.at[pl.ds(roff + row0, _ROWS_SUB)],
                    acc.at[pl.ds(row0, _ROWS_SUB)])

    @pl.when(s == _NS - 1)
    def _():
        pltpu.sync_copy(h_hbm.at[pl.ds(roff + _NS * _ROWS_SUB, _ROWS_TAIL)],
                        acc.at[pl.ds(_NS * _ROWS_SUB, _ROWS_TAIL)])

    plsc.subcore_barrier()

    def idx_descs(g, p, mk):
        eoff = g * _GE
        return [mk(src_hbm.at[pl.ds(src_base + eoff, _CH)],
                   sbufs[p].at[0], isems[p]),
                mk(dst_hbm.at[pl.ds(dst_base + eoff, _CH)],
                   dbuf.at[p, 0], isems[p])]

    def fire_idx(g, p):
        idx_descs(g, p, pltpu.async_copy)

    def wait_idx(g, p):
        for d in idx_descs(g, p, pltpu.make_async_copy):
            d.wait()

    def fire_gather(p):
        pltpu.async_copy(h_hbm.at[sbufs[p].at[0]], rbuf.at[p], gsems[p])

    def wait_gather(p):
        pltpu.make_async_copy(h_hbm.at[sbufs[p].at[0]], rbuf.at[p],
                              gsems[p]).wait()

    def group(g, p, wait_nidx, fire_ngather, fire_idx2):
        if wait_nidx:
            wait_idx(g + 1, 1 - p)
        wait_gather(p)
        if fire_ngather:
            fire_gather(1 - p)
        pltpu.sync_copy(rbuf.at[p], acc.at[dbuf.at[p, 0]], add=True)
        if fire_idx2:
            fire_idx(g + 2, p)

    fire_idx(0, 0)
    wait_idx(0, 0)
    fire_gather(0)
    fire_idx(1, 1)
    group(0, 0, True, True, True)
    group(1, 1, True, True, True)

    def body(t, _):
        group(2 * t, 0, True, True, True)
        group(2 * t + 1, 1, True, True, True)
        return 0

    lax.fori_loop(1, ngrp // 2 - 1, body, 0)
    group(ngrp - 2, 0, True, True, False)
    group(ngrp - 1, 1, False, False, False)

    toff = ngrp * _GE
    pltpu.sync_copy(src_hbm.at[pl.ds(src_base + toff, rem)],
                    sbuf0.at[0, pl.ds(0, rem)])
    pltpu.sync_copy(dst_hbm.at[pl.ds(dst_base + toff, rem)],
                    dbuf.at[0, 0, pl.ds(0, rem)])
    for j in range(rem // 16, _CH // 16):
        sbuf0[0, pl.ds(j * 16, 16)] = jnp.zeros((16,), jnp.int32) + fillv
        dbuf[0, 0, pl.ds(j * 16, 16)] = jnp.full((16,), _N, jnp.int32)
    pltpu.async_copy(h_hbm.at[sbuf0.at[0]], rbuf.at[0], gsem0).wait()
    pltpu.sync_copy(rbuf.at[0], acc.at[dbuf.at[0, 0]], add=True)

    plsc.subcore_barrier()
    pltpu.sync_copy(acc.at[pl.ds(row0, _ROWS_SUB)], out_at(row0, _ROWS_SUB))

    @pl.when(s == _NS - 1)
    def _():
        pltpu.sync_copy(acc.at[pl.ds(_NS * _ROWS_SUB, _ROWS_TAIL)],
                        out_at(_NS * _ROWS_SUB, _ROWS_TAIL))


def _sc_agg_scratch():
    return [
        pltpu.VMEM((1, _CH), jnp.int32),
        pltpu.VMEM((1, _CH), jnp.int32),
        pltpu.VMEM((2, 1, _CH), jnp.int32),
        pltpu.VMEM((2, _CH, _HH), jnp.float32),
        pltpu.VMEM_SHARED((_ACC_PAD, _HH), jnp.float32),
        pltpu.SemaphoreType.DMA,
        pltpu.SemaphoreType.DMA,
        pltpu.SemaphoreType.DMA,
        pltpu.SemaphoreType.DMA,
    ]


@functools.lru_cache(maxsize=None)
def _sc_agg_big_call():
    return pl.kernel(
        functools.partial(_sc_agg_body, True),
        out_type=jax.ShapeDtypeStruct((2 * _N, _HH), jnp.float32),
        mesh=_sc_mesh(),
        scratch_types=_sc_agg_scratch(),
    )


@functools.lru_cache(maxsize=None)
def _sc_agg_small_call():
    return pl.kernel(
        functools.partial(_sc_agg_body, False),
        out_type=jax.ShapeDtypeStruct((_NC, _N, _HH), jnp.float32),
        mesh=_sc_mesh(),
        scratch_types=_sc_agg_scratch(),
    )


_R = 1000
_GRID = _N // _R


def _mlp_tail(t, wa_ref, ba_ref, wb_ref, bb_ref):
    u = lax.dot_general(t, wa_ref[...], (((1,), (1,)), ((), ())),
                        preferred_element_type=jnp.float32,
                        precision=lax.Precision.HIGHEST) + ba_ref[...]
    u = jnp.maximum(u, 0.0)
    return lax.dot_general(u, wb_ref[...], (((1,), (1,)), ((), ())),
                           preferred_element_type=jnp.float32,
                           precision=lax.Precision.HIGHEST) + bb_ref[...]


def _mlp1_body(eps_ref, p_ref, x_ref, wa_ref, ba_ref, wb_ref, bb_ref, o_ref):
    x = x_ref[...]
    agg = p_ref[0][:, :4] + p_ref[1][:, :4] - x
    t = (1.0 + eps_ref[0, 0]) * x + jnp.maximum(agg, 0.0)
    h = _mlp_tail(t, wa_ref, ba_ref, wb_ref, bb_ref)
    o_ref[0] = h[:, :_HH]
    o_ref[1] = h[:, _HH:]


def _mlp_mid_body(eps_ref, s_ref, h_ref, wa_ref, ba_ref, wb_ref, bb_ref, o_ref):
    scale = 1.0 + eps_ref[0, 0]
    t0 = scale * h_ref[0] + jnp.maximum(s_ref[0], 0.0)
    t1 = scale * h_ref[1] + jnp.maximum(s_ref[1], 0.0)
    t = jnp.concatenate([t0, t1], axis=1)
    h = _mlp_tail(t, wa_ref, ba_ref, wb_ref, bb_ref)
    o_ref[0] = h[:, :_HH]
    o_ref[1] = h[:, _HH:]


def _mlp_pool_body(eps_ref, s_ref, h_ref, wa_ref, ba_ref, wb_ref, bb_ref,
                   b_ref, sums_ref):
    @pl.when(pl.program_id(0) == 0)
    def _():
        sums_ref[...] = jnp.zeros_like(sums_ref)

    scale = 1.0 + eps_ref[0, 0]
    t0 = scale * h_ref[0] + jnp.maximum(s_ref[0], 0.0)
    t1 = scale * h_ref[1] + jnp.maximum(s_ref[1], 0.0)
    t = jnp.concatenate([t0, t1], axis=1)
    h = _mlp_tail(t, wa_ref, ba_ref, wb_ref, bb_ref)
    bids = b_ref[0, 0, :]
    onehot = (bids[None, :] == lax.broadcasted_iota(jnp.int32, (_G, _R), 0)
              ).astype(jnp.float32)
    sums_ref[...] += lax.dot_general(onehot, h, (((1,), (0,)), ((), ())),
                                     preferred_element_type=jnp.float32,
                                     precision=lax.Precision.HIGHEST)


def _final_body(sums_ref, b_ref, wfc_ref, o_ref):
    b = b_ref[...]
    onehot = (b[None, :, :] ==
              lax.broadcasted_iota(jnp.int32, (_G,) + b.shape, 0)
              ).astype(jnp.float32)
    counts = jnp.maximum(jnp.sum(onehot, axis=(1, 2)), 1.0)
    pooled = sums_ref[...] / counts[:, None]
    logits = lax.dot_general(pooled, wfc_ref[...], (((1,), (1,)), ((), ())),
                             preferred_element_type=jnp.float32,
                             precision=lax.Precision.HIGHEST)
    o_ref[...] = jax.nn.sigmoid(logits)


_smem11 = pl.BlockSpec((1, 1), lambda i: (0, 0), memory_space=pltpu.SMEM)
_whole = lambda shape: pl.BlockSpec(shape, lambda i: tuple(0 for _ in shape))
_tc_params = pltpu.CompilerParams(dimension_semantics=("arbitrary",))

_mlp1_call = pl.pallas_call(
    _mlp1_body,
    grid=(_GRID,),
    in_specs=[
        _smem11,
        pl.BlockSpec((2, _R, _HH), lambda i: (0, i, 0)),
        pl.BlockSpec((_R, 4), lambda i: (i, 0)),
        _whole((_H, 4)), _whole((1, _H)), _whole((_H, _H)), _whole((1, _H)),
    ],
    out_specs=pl.BlockSpec((2, _R, _HH), lambda i: (0, i, 0)),
    out_shape=jax.ShapeDtypeStruct((2, _N, _HH), jnp.float32),
    compiler_params=_tc_params,
)

_mlp_mid_call = pl.pallas_call(
    _mlp_mid_body,
    grid=(_GRID,),
    in_specs=[
        _smem11,
        pl.BlockSpec((2, _R, _HH), lambda i: (0, i, 0)),
        pl.BlockSpec((2, _R, _HH), lambda i: (0, i, 0)),
        _whole((_H, _H)), _whole((1, _H)), _whole((_H, _H)), _whole((1, _H)),
    ],
    out_specs=pl.BlockSpec((2, _R, _HH), lambda i: (0, i, 0)),
    out_shape=jax.ShapeDtypeStruct((2, _N, _HH), jnp.float32),
    compiler_params=_tc_params,
)

_mlp_pool_call = pl.pallas_call(
    _mlp_pool_body,
    grid=(_GRID,),
    in_specs=[
        _smem11,
        pl.BlockSpec((2, _R, _HH), lambda i: (0, i, 0)),
        pl.BlockSpec((2, _R, _HH), lambda i: (0, i, 0)),
        _whole((_H, _H)), _whole((1, _H)), _whole((_H, _H)), _whole((1, _H)),
        pl.BlockSpec((1, 1, _R), lambda i: (i, 0, 0)),
    ],
    out_specs=pl.BlockSpec((_G, _H), lambda i: (0, 0)),
    out_shape=jax.ShapeDtypeStruct((_G, _H), jnp.float32),
    compiler_params=_tc_params,
)

_final_call = pl.pallas_call(
    _final_body,
    out_shape=jax.ShapeDtypeStruct((_G, 1), jnp.float32),
)


def kernel(x, edge_attr, W1a, b1a, W1b, b1b, eps1, W2a, b2a, W2b, b2b, eps2,
           W3a, b3a, W3b, b3b, eps3, Wfc, edge_index, batch):
    e1 = eps1.reshape(1, 1)
    e2 = eps2.reshape(1, 1)
    e3 = eps3.reshape(1, 1)
    b1a_, b1b_ = b1a.reshape(1, _H), b1b.reshape(1, _H)
    b2a_, b2b_ = b2a.reshape(1, _H), b2b.reshape(1, _H)
    b3a_, b3b_ = b3a.reshape(1, _H), b3b.reshape(1, _H)

    src = edge_index[0]
    dst = edge_index[1]
    zpad = jnp.zeros((_IDX_PAD,), jnp.int32)
    src_p = jnp.concatenate([src, zpad])
    src2_p = jnp.concatenate([src, src + _N, zpad])
    dst_p = jnp.concatenate([dst, zpad])
    x_pad = jnp.pad(x, ((0, 0), (0, _HH - 4)))

    p = _sc_agg_small_call()(x_pad, src_p, dst_p)
    h1 = _mlp1_call(e1, p, x, W1a, b1a_, W1b, b1b_)

    s2 = _sc_agg_big_call()(h1.reshape(2 * _N, _HH), src2_p, dst_p)
    h2 = _mlp_mid_call(e2, s2.reshape(2, _N, _HH), h1, W2a, b2a_, W2b, b2b_)

    s3 = _sc_agg_big_call()(h2.reshape(2 * _N, _HH), src2_p, dst_p)
    sums = _mlp_pool_call(e3, s3.reshape(2, _N, _HH), h2, W3a, b3a_, W3b,
                          b3b_, batch.reshape(_GRID, 1, _R))

    return _final_call(sums, batch.reshape(8, _N // 8), Wfc)


# speedup vs baseline: 6.5880x; 1.0539x over previous
import functools

import jax
import jax.numpy as jnp
from jax import lax
from jax.experimental import pallas as pl
from jax.experimental.pallas import tpu as pltpu, tpu_sc as plsc

_N = 10000
_E = 320000
_H = 256
_HH = 128
_G = 64
_NC = 2
_NS = 16
_CHUNK = 128
_ROWS_SUB = 624
_ROWS_TAIL = _N - _NS * _ROWS_SUB
_ACC_PAD = _N + 8

@functools.lru_cache(maxsize=None)
def _sc_mesh():
    return plsc.VectorSubcoreMesh(core_axis_name="c", subcore_axis_name="s",
                                  num_cores=_NC, num_subcores=_NS)


_CH = 128
_GE = _CH
_IDX_PAD = 512


def _sc_agg_body(big, h_hbm, src_hbm, dst_hbm, out_hbm,
                 sbuf0, sbuf1, sbuf2, dbuf, rbuf, acc,
                 isem0, isem1, isem2, gsem0, gsem1, gsem2):
    c = lax.axis_index("c")
    s = lax.axis_index("s")
    isems = (isem0, isem1, isem2)
    gsems = (gsem0, gsem1, gsem2)
    sbufs = (sbuf0, sbuf1, sbuf2)
    if big:
        epw = _E // _NS
        src_base = c * _E + s * epw
        dst_base = s * epw
        roff = c * _N
        fillv = c * _N
    else:
        epw = _E // (_NC * _NS)
        src_base = c * (_E // _NC) + s * epw
        dst_base = src_base
        roff = 0
        fillv = 0
    ngrp = epw // _GE
    rem = epw - ngrp * _GE

    def out_at(lo, n):
        if big:
            return out_hbm.at[pl.ds(roff + lo, n)]
        return out_hbm.at[c, pl.ds(lo, n)]

    row0 = s * _ROWS_SUB
    pltpu.sync_copy(h_hbm.at[pl.ds(roff + row0, _ROWS_SUB)],
                    acc.at[pl.ds(row0, _ROWS_SUB)])

    @pl.when(s == _NS - 1)
    def _():
        pltpu.sync_copy(h_hbm.at[pl.ds(roff + _NS * _ROWS_SUB, _ROWS_TAIL)],
                        acc.at[pl.ds(_NS * _ROWS_SUB, _ROWS_TAIL)])

    plsc.subcore_barrier()

    def idx_descs(g, p, mk):
        eoff = g * _GE
        return [mk(src_hbm.at[pl.ds(src_base + eoff, _CH)],
                   sbufs[p].at[0], isems[p]),
                mk(dst_hbm.at[pl.ds(dst_base + eoff, _CH)],
                   dbuf.at[p, 0], isems[p])]

    def fire_idx(g, p):
        idx_descs(g, p, pltpu.async_copy)

    def wait_idx(g, p):
        for d in idx_descs(g, p, pltpu.make_async_copy):
            d.wait()

    def fire_gather(p):
        pltpu.async_copy(h_hbm.at[sbufs[p].at[0]], rbuf.at[p], gsems[p])

    def wait_gather(p):
        pltpu.make_async_copy(h_hbm.at[sbufs[p].at[0]], rbuf.at[p],
                              gsems[p]).wait()

    def group(g, p, wait_idx2, fire_g2, fire_i3):
        if wait_idx2:
            wait_idx(g + 2, (p + 2) % 3)
        wait_gather(p)
        if fire_g2:
            fire_gather((p + 2) % 3)
        pltpu.sync_copy(rbuf.at[p], acc.at[dbuf.at[p, 0]], add=True)
        if fire_i3:
            fire_idx(g + 3, p)

    fire_idx(0, 0)
    wait_idx(0, 0)
    fire_gather(0)
    fire_idx(1, 1)
    wait_idx(1, 1)
    fire_gather(1)
    fire_idx(2, 2)

    def body(t, _):
        group(3 * t, 0, True, True, True)
        group(3 * t + 1, 1, True, True, True)
        group(3 * t + 2, 2, True, True, True)
        return 0

    lax.fori_loop(0, ngrp // 3 - 1, body, 0)
    group(ngrp - 3, 0, True, True, False)
    group(ngrp - 2, 1, False, False, False)
    group(ngrp - 1, 2, False, False, False)

    toff = ngrp * _GE
    pltpu.sync_copy(src_hbm.at[pl.ds(src_base + toff, rem)],
                    sbuf0.at[0, pl.ds(0, rem)])
    pltpu.sync_copy(dst_hbm.at[pl.ds(dst_base + toff, rem)],
                    dbuf.at[0, 0, pl.ds(0, rem)])
    for j in range(rem // 16, _CH // 16):
        sbuf0[0, pl.ds(j * 16, 16)] = jnp.zeros((16,), jnp.int32) + fillv
        dbuf[0, 0, pl.ds(j * 16, 16)] = jnp.full((16,), _N, jnp.int32)
    pltpu.async_copy(h_hbm.at[sbuf0.at[0]], rbuf.at[0], gsem0).wait()
    pltpu.sync_copy(rbuf.at[0], acc.at[dbuf.at[0, 0]], add=True)

    plsc.subcore_barrier()
    pltpu.sync_copy(acc.at[pl.ds(row0, _ROWS_SUB)], out_at(row0, _ROWS_SUB))

    @pl.when(s == _NS - 1)
    def _():
        pltpu.sync_copy(acc.at[pl.ds(_NS * _ROWS_SUB, _ROWS_TAIL)],
                        out_at(_NS * _ROWS_SUB, _ROWS_TAIL))


def _sc_agg_scratch():
    return [
        pltpu.VMEM((1, _CH), jnp.int32),
        pltpu.VMEM((1, _CH), jnp.int32),
        pltpu.VMEM((1, _CH), jnp.int32),
        pltpu.VMEM((3, 1, _CH), jnp.int32),
        pltpu.VMEM((3, _CH, _HH), jnp.float32),
        pltpu.VMEM_SHARED((_ACC_PAD, _HH), jnp.float32),
        pltpu.SemaphoreType.DMA,
        pltpu.SemaphoreType.DMA,
        pltpu.SemaphoreType.DMA,
        pltpu.SemaphoreType.DMA,
        pltpu.SemaphoreType.DMA,
        pltpu.SemaphoreType.DMA,
    ]


@functools.lru_cache(maxsize=None)
def _sc_agg_big_call():
    return pl.kernel(
        functools.partial(_sc_agg_body, True),
        out_type=jax.ShapeDtypeStruct((2 * _N, _HH), jnp.float32),
        mesh=_sc_mesh(),
        scratch_types=_sc_agg_scratch(),
    )


@functools.lru_cache(maxsize=None)
def _sc_agg_small_call():
    return pl.kernel(
        functools.partial(_sc_agg_body, False),
        out_type=jax.ShapeDtypeStruct((_NC, _N, _HH), jnp.float32),
        mesh=_sc_mesh(),
        scratch_types=_sc_agg_scratch(),
    )


_R = 1000
_GRID = _N // _R


def _mlp_tail(t, wa_ref, ba_ref, wb_ref, bb_ref):
    u = lax.dot_general(t, wa_ref[...], (((1,), (1,)), ((), ())),
                        preferred_element_type=jnp.float32,
                        precision=lax.Precision.HIGHEST) + ba_ref[...]
    u = jnp.maximum(u, 0.0)
    return lax.dot_general(u, wb_ref[...], (((1,), (1,)), ((), ())),
                           preferred_element_type=jnp.float32,
                           precision=lax.Precision.HIGHEST) + bb_ref[...]


def _mlp1_body(eps_ref, p_ref, x_ref, wa_ref, ba_ref, wb_ref, bb_ref, o_ref):
    x = x_ref[...]
    agg = p_ref[0][:, :4] + p_ref[1][:, :4] - x
    t = (1.0 + eps_ref[0, 0]) * x + jnp.maximum(agg, 0.0)
    h = _mlp_tail(t, wa_ref, ba_ref, wb_ref, bb_ref)
    o_ref[0] = h[:, :_HH]
    o_ref[1] = h[:, _HH:]


def _mlp_mid_body(eps_ref, s_ref, h_ref, wa_ref, ba_ref, wb_ref, bb_ref, o_ref):
    scale = 1.0 + eps_ref[0, 0]
    t0 = scale * h_ref[0] + jnp.maximum(s_ref[0], 0.0)
    t1 = scale * h_ref[1] + jnp.maximum(s_ref[1], 0.0)
    t = jnp.concatenate([t0, t1], axis=1)
    h = _mlp_tail(t, wa_ref, ba_ref, wb_ref, bb_ref)
    o_ref[0] = h[:, :_HH]
    o_ref[1] = h[:, _HH:]


def _mlp_pool_body(eps_ref, s_ref, h_ref, wa_ref, ba_ref, wb_ref, bb_ref,
                   b_ref, sums_ref):
    @pl.when(pl.program_id(0) == 0)
    def _():
        sums_ref[...] = jnp.zeros_like(sums_ref)

    scale = 1.0 + eps_ref[0, 0]
    t0 = scale * h_ref[0] + jnp.maximum(s_ref[0], 0.0)
    t1 = scale * h_ref[1] + jnp.maximum(s_ref[1], 0.0)
    t = jnp.concatenate([t0, t1], axis=1)
    h = _mlp_tail(t, wa_ref, ba_ref, wb_ref, bb_ref)
    bids = b_ref[0, 0, :]
    onehot = (bids[None, :] == lax.broadcasted_iota(jnp.int32, (_G, _R), 0)
              ).astype(jnp.float32)
    sums_ref[...] += lax.dot_general(onehot, h, (((1,), (0,)), ((), ())),
                                     preferred_element_type=jnp.float32,
                                     precision=lax.Precision.HIGHEST)


def _final_body(sums_ref, b_ref, wfc_ref, o_ref):
    b = b_ref[...]
    onehot = (b[None, :, :] ==
              lax.broadcasted_iota(jnp.int32, (_G,) + b.shape, 0)
              ).astype(jnp.float32)
    counts = jnp.maximum(jnp.sum(onehot, axis=(1, 2)), 1.0)
    pooled = sums_ref[...] / counts[:, None]
    logits = lax.dot_general(pooled, wfc_ref[...], (((1,), (1,)), ((), ())),
                             preferred_element_type=jnp.float32,
                             precision=lax.Precision.HIGHEST)
    o_ref[...] = jax.nn.sigmoid(logits)


_smem11 = pl.BlockSpec((1, 1), lambda i: (0, 0), memory_space=pltpu.SMEM)
_whole = lambda shape: pl.BlockSpec(shape, lambda i: tuple(0 for _ in shape))
_tc_params = pltpu.CompilerParams(dimension_semantics=("arbitrary",))

_mlp1_call = pl.pallas_call(
    _mlp1_body,
    grid=(_GRID,),
    in_specs=[
        _smem11,
        pl.BlockSpec((2, _R, _HH), lambda i: (0, i, 0)),
        pl.BlockSpec((_R, 4), lambda i: (i, 0)),
        _whole((_H, 4)), _whole((1, _H)), _whole((_H, _H)), _whole((1, _H)),
    ],
    out_specs=pl.BlockSpec((2, _R, _HH), lambda i: (0, i, 0)),
    out_shape=jax.ShapeDtypeStruct((2, _N, _HH), jnp.float32),
    compiler_params=_tc_params,
)

_mlp_mid_call = pl.pallas_call(
    _mlp_mid_body,
    grid=(_GRID,),
    in_specs=[
        _smem11,
        pl.BlockSpec((2, _R, _HH), lambda i: (0, i, 0)),
        pl.BlockSpec((2, _R, _HH), lambda i: (0, i, 0)),
        _whole((_H, _H)), _whole((1, _H)), _whole((_H, _H)), _whole((1, _H)),
    ],
    out_specs=pl.BlockSpec((2, _R, _HH), lambda i: (0, i, 0)),
    out_shape=jax.ShapeDtypeStruct((2, _N, _HH), jnp.float32),
    compiler_params=_tc_params,
)

_mlp_pool_call = pl.pallas_call(
    _mlp_pool_body,
    grid=(_GRID,),
    in_specs=[
        _smem11,
        pl.BlockSpec((2, _R, _HH), lambda i: (0, i, 0)),
        pl.BlockSpec((2, _R, _HH), lambda i: (0, i, 0)),
        _whole((_H, _H)), _whole((1, _H)), _whole((_H, _H)), _whole((1, _H)),
        pl.BlockSpec((1, 1, _R), lambda i: (i, 0, 0)),
    ],
    out_specs=pl.BlockSpec((_G, _H), lambda i: (0, 0)),
    out_shape=jax.ShapeDtypeStruct((_G, _H), jnp.float32),
    compiler_params=_tc_params,
)

_final_call = pl.pallas_call(
    _final_body,
    out_shape=jax.ShapeDtypeStruct((_G, 1), jnp.float32),
)


def kernel(x, edge_attr, W1a, b1a, W1b, b1b, eps1, W2a, b2a, W2b, b2b, eps2,
           W3a, b3a, W3b, b3b, eps3, Wfc, edge_index, batch):
    e1 = eps1.reshape(1, 1)
    e2 = eps2.reshape(1, 1)
    e3 = eps3.reshape(1, 1)
    b1a_, b1b_ = b1a.reshape(1, _H), b1b.reshape(1, _H)
    b2a_, b2b_ = b2a.reshape(1, _H), b2b.reshape(1, _H)
    b3a_, b3b_ = b3a.reshape(1, _H), b3b.reshape(1, _H)

    src = edge_index[0]
    dst = edge_index[1]
    zpad = jnp.zeros((_IDX_PAD,), jnp.int32)
    src_p = jnp.concatenate([src, zpad])
    src2_p = jnp.concatenate([src, src + _N, zpad])
    dst_p = jnp.concatenate([dst, zpad])
    x_pad = jnp.pad(x, ((0, 0), (0, _HH - 4)))

    p = _sc_agg_small_call()(x_pad, src_p, dst_p)
    h1 = _mlp1_call(e1, p, x, W1a, b1a_, W1b, b1b_)

    s2 = _sc_agg_big_call()(h1.reshape(2 * _N, _HH), src2_p, dst_p)
    h2 = _mlp_mid_call(e2, s2.reshape(2, _N, _HH), h1, W2a, b2a_, W2b, b2b_)

    s3 = _sc_agg_big_call()(h2.reshape(2 * _N, _HH), src2_p, dst_p)
    sums = _mlp_pool_call(e3, s3.reshape(2, _N, _HH), h2, W3a, b3a_, W3b,
                          b3b_, batch.reshape(_GRID, 1, _R))

    return _final_call(sums, batch.reshape(8, _N // 8), Wfc)
